# baseline (speedup 1.0000x reference)
import functools

import jax
import jax.numpy as jnp
from jax import lax
from jax.experimental import pallas as pl
from jax.experimental.pallas import tpu as pltpu
from jax.experimental.pallas import tpu_sc as plsc

NC = 2
NS = 16
NW = NC * NS
L = 16

D = 128
CHUNK = 80


def _make_sc_call(n_edges):
    e_per = n_edges // NW
    n_ch = e_per // CHUNK
    assert e_per * NW == n_edges and n_ch * CHUNK == e_per and n_ch % 2 == 1

    mesh = plsc.VectorSubcoreMesh(
        core_axis_name="c", subcore_axis_name="s",
        num_cores=NC, num_subcores=NS)

    @functools.partial(
        pl.kernel,
        out_type=jax.ShapeDtypeStruct((n_edges,), jnp.float32),
        mesh=mesh,
        scratch_types=[
            pltpu.VMEM((e_per,), jnp.int32),
            pltpu.VMEM((e_per,), jnp.int32),
            pltpu.VMEM((CHUNK, D), jnp.float32),
            pltpu.VMEM((CHUNK, D), jnp.float32),
            pltpu.VMEM((CHUNK, D), jnp.float32),
            pltpu.VMEM((CHUNK, D), jnp.float32),
            pltpu.VMEM((e_per,), jnp.float32),
            pltpu.SemaphoreType.DMA,
            pltpu.SemaphoreType.DMA,
            pltpu.SemaphoreType.DMA,
            pltpu.SemaphoreType.DMA,
        ],
    )
    def sc_call(xu_hbm, xb_hbm, src_hbm, dst_hbm, out_hbm,
                idx_u, idx_b, ru_a, rb_a, ru_b, rb_b, out_v,
                sem_ua, sem_ba, sem_ub, sem_bb):
        cid = lax.axis_index("c")
        sid = lax.axis_index("s")
        wid = sid * NC + cid
        base = wid * e_per

        lane = lax.iota(jnp.int32, 16)
        perms = [(lane + sh) % 16 for sh in (8, 4, 2, 1)]

        pltpu.sync_copy(src_hbm.at[pl.ds(base, e_per)], idx_u)
        pltpu.sync_copy(dst_hbm.at[pl.ds(base, e_per)], idx_b)

        def gathers(c, ru, rb, su, sb):
            iu = idx_u.at[pl.ds(c * CHUNK, CHUNK)]
            ib = idx_b.at[pl.ds(c * CHUNK, CHUNK)]
            return (pltpu.make_async_copy(xu_hbm.at[iu], ru, su),
                    pltpu.make_async_copy(xb_hbm.at[ib], rb, sb))

        def issue(c, ru, rb, su, sb):
            for cp in gathers(c, ru, rb, su, sb):
                cp.start()

        def wait(c, ru, rb, su, sb):
            for cp in gathers(c, ru, rb, su, sb):
                cp.wait()

        def compute(c, ru, rb):
            @plsc.parallel_loop(0, CHUNK // L, unroll=1)
            def group_body(g):
                res = jnp.zeros((L,), jnp.float32)
                for j in range(L):
                    e = g * L + j
                    acc = ru[e, pl.ds(0, L)] * rb[e, pl.ds(0, L)]
                    for d in range(1, D // L):
                        acc = acc + (ru[e, pl.ds(d * L, L)]
                                     * rb[e, pl.ds(d * L, L)])
                    for p in perms:
                        acc = acc + acc.at[p].get(mode="promise_in_bounds")
                    res = jnp.where(lane == j, acc, res)
                out_v[pl.ds(c * CHUNK + g * L, L)] = res

        buf_a = (ru_a, rb_a, sem_ua, sem_ba)
        buf_b = (ru_b, rb_b, sem_ub, sem_bb)

        issue(0, *buf_a)

        def pair_body(p, carry):
            c0 = 2 * p
            issue(c0 + 1, *buf_b)
            wait(c0, *buf_a)
            compute(c0, ru_a, rb_a)
            issue(c0 + 2, *buf_a)
            wait(c0 + 1, *buf_b)
            compute(c0 + 1, ru_b, rb_b)
            return carry

        lax.fori_loop(0, (n_ch - 1) // 2, pair_body, 0)

        last = n_ch - 1
        wait(last, *buf_a)
        compute(last, ru_a, rb_a)

        pltpu.sync_copy(out_v, out_hbm.at[pl.ds(base, e_per)])

    return sc_call


def kernel(x_user, x_business, edge_label_index):
    n_edges = edge_label_index.shape[1]
    idx = edge_label_index.astype(jnp.int32)

    sc_call = _make_sc_call(n_edges)
    return sc_call(x_user, x_business, idx[0], idx[1])


# speedup vs baseline: 2.1025x; 2.1025x over previous
import functools

import jax
import jax.numpy as jnp
from jax import lax
from jax.experimental import pallas as pl
from jax.experimental.pallas import tpu as pltpu
from jax.experimental.pallas import tpu_sc as plsc

NC = 2
NS = 16
NW = NC * NS
L = 16

D = 128
CHUNK = 80


def _make_sc_call(n_edges):
    e_per = n_edges // NW
    n_ch = e_per // CHUNK
    assert e_per * NW == n_edges and n_ch * CHUNK == e_per and n_ch % 2 == 1

    mesh = plsc.VectorSubcoreMesh(
        core_axis_name="c", subcore_axis_name="s",
        num_cores=NC, num_subcores=NS)

    @functools.partial(
        pl.kernel,
        out_type=jax.ShapeDtypeStruct((n_edges,), jnp.float32),
        mesh=mesh,
        scratch_types=[
            pltpu.VMEM((e_per,), jnp.int32),
            pltpu.VMEM((e_per,), jnp.int32),
            pltpu.VMEM((CHUNK, D), jnp.float32),
            pltpu.VMEM((CHUNK, D), jnp.float32),
            pltpu.VMEM((CHUNK, D), jnp.float32),
            pltpu.VMEM((CHUNK, D), jnp.float32),
            pltpu.VMEM((e_per,), jnp.float32),
            pltpu.SemaphoreType.DMA,
            pltpu.SemaphoreType.DMA,
            pltpu.SemaphoreType.DMA,
            pltpu.SemaphoreType.DMA,
        ],
    )
    def sc_call(xu_hbm, xb_hbm, src_hbm, dst_hbm, out_hbm,
                idx_u, idx_b, ru_a, rb_a, ru_b, rb_b, out_v,
                sem_ua, sem_ba, sem_ub, sem_bb):
        cid = lax.axis_index("c")
        sid = lax.axis_index("s")
        wid = sid * NC + cid
        base = wid * e_per

        lane = lax.iota(jnp.int32, 16)
        perms = [(lane + sh) % 16 for sh in (8, 4, 2, 1)]

        pltpu.sync_copy(src_hbm.at[pl.ds(base, e_per)], idx_u)
        pltpu.sync_copy(dst_hbm.at[pl.ds(base, e_per)], idx_b)

        def gathers(c, ru, rb, su, sb):
            iu = idx_u.at[pl.ds(c * CHUNK, CHUNK)]
            ib = idx_b.at[pl.ds(c * CHUNK, CHUNK)]
            return (pltpu.make_async_copy(xu_hbm.at[iu], ru, su),
                    pltpu.make_async_copy(xb_hbm.at[ib], rb, sb))

        def issue(c, ru, rb, su, sb):
            for cp in gathers(c, ru, rb, su, sb):
                cp.start()

        def wait(c, ru, rb, su, sb):
            for cp in gathers(c, ru, rb, su, sb):
                cp.wait()

        def compute(c, ru, rb):
            def group_body(g, carry):
                e0 = g * L

                def d_body(d, accs):
                    return tuple(
                        accs[j] + (ru[e0 + j, pl.ds(d * L, L)]
                                   * rb[e0 + j, pl.ds(d * L, L)])
                        for j in range(L))

                accs = tuple(jnp.zeros((L,), jnp.float32) for _ in range(L))
                accs = lax.fori_loop(0, D // L, d_body, accs)
                res = jnp.zeros((L,), jnp.float32)
                for j in range(L):
                    a = accs[j]
                    for p in perms:
                        a = a + a.at[p].get(mode="promise_in_bounds")
                    res = jnp.where(lane == j, a, res)
                out_v[pl.ds(c * CHUNK + e0, L)] = res
                return carry

            lax.fori_loop(0, CHUNK // L, group_body, 0)

        buf_a = (ru_a, rb_a, sem_ua, sem_ba)
        buf_b = (ru_b, rb_b, sem_ub, sem_bb)

        issue(0, *buf_a)

        def pair_body(p, carry):
            c0 = 2 * p
            issue(c0 + 1, *buf_b)
            wait(c0, *buf_a)
            compute(c0, ru_a, rb_a)
            issue(c0 + 2, *buf_a)
            wait(c0 + 1, *buf_b)
            compute(c0 + 1, ru_b, rb_b)
            return carry

        lax.fori_loop(0, (n_ch - 1) // 2, pair_body, 0)

        last = n_ch - 1
        wait(last, *buf_a)
        compute(last, ru_a, rb_a)

        pltpu.sync_copy(out_v, out_hbm.at[pl.ds(base, e_per)])

    return sc_call


def kernel(x_user, x_business, edge_label_index):
    n_edges = edge_label_index.shape[1]
    idx = edge_label_index.astype(jnp.int32)

    sc_call = _make_sc_call(n_edges)
    return sc_call(x_user, x_business, idx[0], idx[1])


# speedup vs baseline: 2.1310x; 1.0136x over previous
import functools

import jax
import jax.numpy as jnp
from jax import lax
from jax.experimental import pallas as pl
from jax.experimental.pallas import tpu as pltpu
from jax.experimental.pallas import tpu_sc as plsc

NC = 2
NS = 16
NW = NC * NS
L = 16

D = 128
CHUNK = 80


def _make_sc_call(n_edges):
    e_per = n_edges // NW
    n_ch = e_per // CHUNK
    assert e_per * NW == n_edges and n_ch * CHUNK == e_per and n_ch % 2 == 1

    mesh = plsc.VectorSubcoreMesh(
        core_axis_name="c", subcore_axis_name="s",
        num_cores=NC, num_subcores=NS)

    @functools.partial(
        pl.kernel,
        out_type=jax.ShapeDtypeStruct((n_edges,), jnp.float32),
        mesh=mesh,
        scratch_types=[
            pltpu.VMEM((e_per,), jnp.int32),
            pltpu.VMEM((e_per,), jnp.int32),
            pltpu.VMEM((CHUNK, D), jnp.float32),
            pltpu.VMEM((CHUNK, D), jnp.float32),
            pltpu.VMEM((CHUNK, D), jnp.float32),
            pltpu.VMEM((CHUNK, D), jnp.float32),
            pltpu.VMEM((e_per,), jnp.float32),
            pltpu.SemaphoreType.DMA,
            pltpu.SemaphoreType.DMA,
            pltpu.SemaphoreType.DMA,
            pltpu.SemaphoreType.DMA,
        ],
    )
    def sc_call(xu_hbm, xb_hbm, src_hbm, dst_hbm, out_hbm,
                idx_u, idx_b, ru_a, rb_a, ru_b, rb_b, out_v,
                sem_ua, sem_ba, sem_ub, sem_bb):
        cid = lax.axis_index("c")
        sid = lax.axis_index("s")
        wid = sid * NC + cid
        base = wid * e_per

        lane = lax.iota(jnp.int32, 16)
        fold_perms = [lane ^ sh for sh in (8, 4, 2, 1)]
        fold_masks = [(lane & sh) == 0 for sh in (8, 4, 2, 1)]

        pltpu.sync_copy(src_hbm.at[pl.ds(base, e_per)], idx_u)
        pltpu.sync_copy(dst_hbm.at[pl.ds(base, e_per)], idx_b)

        def gathers(c, ru, rb, su, sb):
            iu = idx_u.at[pl.ds(c * CHUNK, CHUNK)]
            ib = idx_b.at[pl.ds(c * CHUNK, CHUNK)]
            return (pltpu.make_async_copy(xu_hbm.at[iu], ru, su),
                    pltpu.make_async_copy(xb_hbm.at[ib], rb, sb))

        def issue(c, ru, rb, su, sb):
            for cp in gathers(c, ru, rb, su, sb):
                cp.start()

        def wait(c, ru, rb, su, sb):
            for cp in gathers(c, ru, rb, su, sb):
                cp.wait()

        def compute(c, ru, rb):
            def group_body(g, carry):
                e0 = g * L

                def d_body(d, accs):
                    return tuple(
                        accs[j] + (ru[e0 + j, pl.ds(d * L, L)]
                                   * rb[e0 + j, pl.ds(d * L, L)])
                        for j in range(L))

                accs = tuple(jnp.zeros((L,), jnp.float32) for _ in range(L))
                accs = lax.fori_loop(0, D // L, d_body, accs)

                def fold(a, p):
                    return a + a.at[p].get(mode="promise_in_bounds")

                vecs = list(accs)
                for p, m in zip(fold_perms, fold_masks):
                    half = len(vecs) // 2
                    vecs = [jnp.where(m, fold(vecs[j], p),
                                      fold(vecs[j + half], p))
                            for j in range(half)]
                out_v[pl.ds(c * CHUNK + e0, L)] = vecs[0]
                return carry

            lax.fori_loop(0, CHUNK // L, group_body, 0)

        buf_a = (ru_a, rb_a, sem_ua, sem_ba)
        buf_b = (ru_b, rb_b, sem_ub, sem_bb)

        issue(0, *buf_a)

        def pair_body(p, carry):
            c0 = 2 * p
            issue(c0 + 1, *buf_b)
            wait(c0, *buf_a)
            compute(c0, ru_a, rb_a)
            issue(c0 + 2, *buf_a)
            wait(c0 + 1, *buf_b)
            compute(c0 + 1, ru_b, rb_b)
            return carry

        lax.fori_loop(0, (n_ch - 1) // 2, pair_body, 0)

        last = n_ch - 1
        wait(last, *buf_a)
        compute(last, ru_a, rb_a)

        pltpu.sync_copy(out_v, out_hbm.at[pl.ds(base, e_per)])

    return sc_call


def kernel(x_user, x_business, edge_label_index):
    n_edges = edge_label_index.shape[1]
    idx = edge_label_index.astype(jnp.int32)

    sc_call = _make_sc_call(n_edges)
    return sc_call(x_user, x_business, idx[0], idx[1])


# speedup vs baseline: 2.3751x; 1.1145x over previous
import functools

import jax
import jax.numpy as jnp
from jax import lax
from jax.experimental import pallas as pl
from jax.experimental.pallas import tpu as pltpu
from jax.experimental.pallas import tpu_sc as plsc

NC = 2
NS = 16
NW = NC * NS
L = 16

D = 128
CHUNK = 80


def _make_sc_call(n_edges, n_nodes):
    e_per = n_edges // NW
    n_ch = e_per // CHUNK
    assert e_per * NW == n_edges and n_ch * CHUNK == e_per and n_ch % 2 == 1
    stripe = -(-(-(-n_nodes // NS)) // 8) * 8
    assert n_nodes % 8 == 0 and n_nodes >= stripe

    mesh = plsc.VectorSubcoreMesh(
        core_axis_name="c", subcore_axis_name="s",
        num_cores=NC, num_subcores=NS)

    @functools.partial(
        pl.kernel,
        out_type=jax.ShapeDtypeStruct((n_edges,), jnp.float32),
        mesh=mesh,
        scratch_types=[
            pltpu.VMEM_SHARED((n_nodes, D), jnp.float32),
            pltpu.VMEM((CHUNK,), jnp.int32),
            pltpu.VMEM((CHUNK,), jnp.int32),
            pltpu.VMEM((CHUNK,), jnp.int32),
            pltpu.VMEM((CHUNK,), jnp.int32),
            pltpu.VMEM((CHUNK, D), jnp.float32),
            pltpu.VMEM((CHUNK, D), jnp.float32),
            pltpu.VMEM((CHUNK, D), jnp.float32),
            pltpu.VMEM((CHUNK, D), jnp.float32),
            pltpu.VMEM((CHUNK,), jnp.float32),
            pltpu.VMEM((CHUNK,), jnp.float32),
            pltpu.SemaphoreType.DMA,
            pltpu.SemaphoreType.DMA,
            pltpu.SemaphoreType.DMA,
            pltpu.SemaphoreType.DMA,
            pltpu.SemaphoreType.DMA,
            pltpu.SemaphoreType.DMA,
            pltpu.SemaphoreType.DMA,
            pltpu.SemaphoreType.DMA,
        ],
    )
    def sc_call(xu_hbm, xb_hbm, src_hbm, dst_hbm, out_hbm,
                xu_sp, iu_a, ib_a, iu_b, ib_b, ru_a, rb_a, ru_b, rb_b,
                ov_a, ov_b,
                sem_ua, sem_ba, sem_ub, sem_bb, semi_a, semi_b,
                semo_a, semo_b):
        cid = lax.axis_index("c")
        sid = lax.axis_index("s")
        wid = sid * NC + cid
        base = wid * e_per

        lane = lax.iota(jnp.int32, 16)
        fold_perms = [lane ^ sh for sh in (8, 4, 2, 1)]
        fold_masks = [(lane & sh) == 0 for sh in (8, 4, 2, 1)]

        r0 = jnp.minimum(sid * stripe, n_nodes - stripe)
        pltpu.sync_copy(xu_hbm.at[pl.ds(r0, stripe)],
                        xu_sp.at[pl.ds(r0, stripe)])
        plsc.subcore_barrier()

        def idx_copies(c, iu, ib, sem):
            off = base + c * CHUNK
            return (pltpu.make_async_copy(src_hbm.at[pl.ds(off, CHUNK)],
                                          iu, sem),
                    pltpu.make_async_copy(dst_hbm.at[pl.ds(off, CHUNK)],
                                          ib, sem))

        def gathers(iu, ib, ru, rb, su, sb):
            return (pltpu.make_async_copy(xu_sp.at[iu], ru, su),
                    pltpu.make_async_copy(xb_hbm.at[ib], rb, sb))

        def out_copy(c, ov, sem):
            return pltpu.make_async_copy(
                ov, out_hbm.at[pl.ds(base + c * CHUNK, CHUNK)], sem)

        def compute(ru, rb, ov):
            def group_body(g, carry):
                e0 = g * L

                def d_body(d, accs):
                    return tuple(
                        accs[j] + (ru[e0 + j, pl.ds(d * L, L)]
                                   * rb[e0 + j, pl.ds(d * L, L)])
                        for j in range(L))

                accs = tuple(jnp.zeros((L,), jnp.float32) for _ in range(L))
                accs = lax.fori_loop(0, D // L, d_body, accs)

                def fold(a, p):
                    return a + a.at[p].get(mode="promise_in_bounds")

                vecs = list(accs)
                for p, m in zip(fold_perms, fold_masks):
                    half = len(vecs) // 2
                    vecs = [jnp.where(m, fold(vecs[j], p),
                                      fold(vecs[j + half], p))
                            for j in range(half)]
                ov[pl.ds(e0, L)] = vecs[0]
                return carry

            lax.fori_loop(0, CHUNK // L, group_body, 0)

        buf_a = (iu_a, ib_a, ru_a, rb_a, ov_a,
                 sem_ua, sem_ba, semi_a, semo_a)
        buf_b = (iu_b, ib_b, ru_b, rb_b, ov_b,
                 sem_ub, sem_bb, semi_b, semo_b)

        def stage(c, buf, nxt_idx_c):
            iu, ib, ru, rb, ov, su, sb, si, so = buf
            for cp in gathers(iu, ib, ru, rb, su, sb):
                cp.wait()
            for cp in idx_copies(nxt_idx_c, iu, ib, si):
                cp.start()
            out_copy(c, ov, so).wait()
            compute(ru, rb, ov)
            out_copy(c, ov, so).start()

        def issue_rows(c_idx_ready, buf):
            iu, ib, ru, rb, ov, su, sb, si, so = buf
            for cp in idx_copies(c_idx_ready, iu, ib, si):
                cp.wait()
            for cp in gathers(iu, ib, ru, rb, su, sb):
                cp.start()

        iu, ib, ru, rb, ov, su, sb, si, so = buf_a
        for cp in idx_copies(0, iu, ib, si):
            cp.start()
        issue_rows(0, buf_a)
        iu, ib, ru, rb, ov, su, sb, si, so = buf_b
        for cp in idx_copies(1, iu, ib, si):
            cp.start()
        out_copy(0, ov_a, semo_a).start()
        out_copy(1, ov_b, semo_b).start()

        def pair_body(p, carry):
            c0 = 2 * p
            issue_rows(c0 + 1, buf_b)
            stage(c0, buf_a, c0 + 2)
            issue_rows(c0 + 2, buf_a)
            stage(c0 + 1, buf_b, jnp.minimum(c0 + 3, n_ch - 1))
            return carry

        lax.fori_loop(0, (n_ch - 1) // 2, pair_body, 0)

        stage(n_ch - 1, buf_a, 0)
        for cp in idx_copies(0, iu_a, ib_a, semi_a):
            cp.wait()
        for cp in idx_copies(0, iu_b, ib_b, semi_b):
            cp.wait()
        out_copy(n_ch - 1, ov_a, semo_a).wait()
        out_copy(n_ch - 2, ov_b, semo_b).wait()

    return sc_call


def kernel(x_user, x_business, edge_label_index):
    n_edges = edge_label_index.shape[1]
    idx = edge_label_index.astype(jnp.int32)

    sc_call = _make_sc_call(n_edges, x_user.shape[0])
    return sc_call(x_user, x_business, idx[0], idx[1])
